# Initial kernel scaffold; baseline (speedup 1.0000x reference)
import jax
import jax.numpy as jnp
from jax.experimental import pallas as pl


def kernel(input_features, edge_index, W1, b1, W2, b2, W3, b3):
    raise NotImplementedError("write your pallas kernel here")


# speedup vs baseline: 7.6702x; 7.6702x over previous
import functools

import jax
import jax.numpy as jnp
from jax import lax
from jax.experimental import pallas as pl
from jax.experimental.pallas import tpu as pltpu
from jax.experimental.pallas import tpu_sc as plsc

NC = 2
NS = 16
C = 125


def _mesh():
    return plsc.VectorSubcoreMesh(
        core_axis_name="c", subcore_axis_name="s", num_cores=NC, num_subcores=NS
    )


def _deg_call(srcr, dstr, zpad, n_nodes, npad):
    rows = srcr.shape[0]
    rpw = rows // (NC * NS)
    G = 16
    assert rpw % G == 0

    @functools.partial(
        pl.kernel,
        out_type=jax.ShapeDtypeStruct((NC * 2 * npad,), jnp.float32),
        mesh=_mesh(),
        scratch_types=[
            pltpu.VMEM((G, C), jnp.int32),
            pltpu.VMEM((G, C), jnp.int32),
            pltpu.VMEM((128,), jnp.float32),
            pltpu.VMEM_SHARED((npad,), jnp.float32),
            pltpu.VMEM_SHARED((npad,), jnp.float32),
        ],
    )
    def deg_kernel(src_hbm, dst_hbm, z_hbm, out_hbm, idxs, idxd, ones, acc_s, acc_d):
        c = lax.axis_index("c")
        s = lax.axis_index("s")
        w = s * NC + c
        for k in range(8):
            ones[pl.ds(k * 16, 16)] = jnp.ones((16,), jnp.float32)
        zlen = npad // NS
        pltpu.sync_copy(z_hbm.at[pl.ds(0, zlen)], acc_s.at[pl.ds(s * zlen, zlen)])
        pltpu.sync_copy(z_hbm.at[pl.ds(0, zlen)], acc_d.at[pl.ds(s * zlen, zlen)])
        plsc.subcore_barrier()

        def group(g, carry):
            pltpu.sync_copy(src_hbm.at[pl.ds(w * rpw + g * G, G)], idxs)
            pltpu.sync_copy(dst_hbm.at[pl.ds(w * rpw + g * G, G)], idxd)

            def body(j, carry2):
                pltpu.sync_copy(ones.at[pl.ds(0, C)], acc_s.at[idxs.at[j]], add=True)
                pltpu.sync_copy(ones.at[pl.ds(0, C)], acc_d.at[idxd.at[j]], add=True)
                return carry2

            return lax.fori_loop(0, G, body, carry)

        lax.fori_loop(0, rpw // G, group, 0)
        plsc.subcore_barrier()

        @pl.when(s < 8)
        def _():
            span = npad // 8
            pltpu.sync_copy(
                acc_s.at[pl.ds(s * span, span)],
                out_hbm.at[pl.ds(c * 2 * npad + s * span, span)],
            )
            pltpu.sync_copy(
                acc_d.at[pl.ds(s * span, span)],
                out_hbm.at[pl.ds(c * 2 * npad + npad + s * span, span)],
            )

    return deg_kernel(srcr, dstr, zpad)


def _mp_call(x_hbm, srcr, dstr, z_hbm, n_nodes, npad, split_cols):
    rows = dstr.shape[0]
    rpw = rows // NS if split_cols else rows // (NC * NS)
    rpn = npad // NS
    G = 16
    assert rpw % G == 0

    @functools.partial(
        pl.kernel,
        out_type=jax.ShapeDtypeStruct((NC * npad, 128), jnp.float32),
        mesh=_mesh(),
        scratch_types=[
            pltpu.VMEM((G, C), jnp.int32),
            pltpu.VMEM((G, C), jnp.int32),
            pltpu.VMEM((C, 128), jnp.float32),
            pltpu.SemaphoreType.DMA,
            pltpu.VMEM_SHARED((npad, 128), jnp.float32),
        ],
    )
    def mp_kernel(xt, src_hbm, dst_hbm, z, out_hbm, idxs, idxd, buf, sem, acc):
        c = lax.axis_index("c")
        s = lax.axis_index("s")
        pltpu.sync_copy(z.at[pl.ds(s * rpn, rpn), :], acc.at[pl.ds(s * rpn, rpn), :])
        plsc.subcore_barrier()
        if split_cols:
            base = c * rows + s * rpw
        else:
            base = (s * NC + c) * rpw
        dbase = s * rpw if split_cols else base

        def group(g, carry):
            pltpu.sync_copy(src_hbm.at[pl.ds(base + g * G, G)], idxs)
            pltpu.sync_copy(dst_hbm.at[pl.ds(dbase + g * G, G)], idxd)

            def body(j, carry2):
                pltpu.async_copy(xt.at[idxs.at[j]], buf, sem).wait()
                pltpu.sync_copy(buf, acc.at[idxd.at[j]], add=True)
                return carry2

            return lax.fori_loop(0, G, body, carry)

        lax.fori_loop(0, rpw // G, group, 0)
        plsc.subcore_barrier()
        pltpu.sync_copy(
            acc.at[pl.ds(s * rpn, rpn), :],
            out_hbm.at[pl.ds(c * npad + s * rpn, rpn), :],
        )

    return mp_kernel(x_hbm, srcr, dstr, z_hbm)


def _prep_call(x, deg4, n, npad):
    def body(x_ref, d_ref, xs_ref, ns_ref, nd_ref):
        d = d_ref[...]
        ds_sum = d[0, 0] + d[1, 0]
        dd_sum = d[0, 1] + d[1, 1]
        ns = lax.rsqrt(jnp.clip(ds_sum, 1.0, None))[:n]
        nd = lax.rsqrt(jnp.clip(dd_sum, 1.0, None))[:n]
        ns_ref[...] = ns
        nd_ref[...] = nd
        xs_ref[...] = x_ref[...] * ns

    return pl.pallas_call(
        body,
        out_shape=(
            jax.ShapeDtypeStruct((n, 128), jnp.float32),
            jax.ShapeDtypeStruct((n, 1), jnp.float32),
            jax.ShapeDtypeStruct((n, 1), jnp.float32),
        ),
    )(x, deg4)


def _mm1_call(p1, w1, b1, ns, nd, n):
    bm = 2000

    def body(p_ref, w_ref, b_ref, nd_ref, ns_ref, o_ref):
        p = p_ref[...]
        agg = (p[0] + p[1]) * nd_ref[...]
        h = jnp.dot(agg, w_ref[...], preferred_element_type=jnp.float32)
        h = jnp.maximum(h + b_ref[...], 0.0) * ns_ref[...]
        o_ref[...] = h[None]

    grid = (n // bm, 2)
    return pl.pallas_call(
        body,
        grid=grid,
        in_specs=[
            pl.BlockSpec((2, bm, 128), lambda i, j: (0, i, 0)),
            pl.BlockSpec((128, 128), lambda i, j: (0, j)),
            pl.BlockSpec((1, 128), lambda i, j: (0, j)),
            pl.BlockSpec((bm, 1), lambda i, j: (i, 0)),
            pl.BlockSpec((bm, 1), lambda i, j: (i, 0)),
        ],
        out_specs=pl.BlockSpec((1, bm, 128), lambda i, j: (j, i, 0)),
        out_shape=jax.ShapeDtypeStruct((2, n, 128), jnp.float32),
    )(p1, w1, b1, nd, ns)


def _mm2_call(p2, w2, b2, w3, ns, nd, n):
    bm = 2000

    def body(p_ref, w2_ref, b2_ref, w3_ref, nd_ref, ns_ref, o_ref):
        p = p_ref[...]
        agg = jnp.concatenate([p[0], p[1]], axis=1) * nd_ref[...]
        h = jnp.dot(agg, w2_ref[...], preferred_element_type=jnp.float32)
        h = jnp.maximum(h + b2_ref[...], 0.0) * ns_ref[...]
        o_ref[...] = jnp.dot(h, w3_ref[...], preferred_element_type=jnp.float32)

    return pl.pallas_call(
        body,
        grid=(n // bm,),
        in_specs=[
            pl.BlockSpec((2, bm, 128), lambda i: (0, i, 0)),
            pl.BlockSpec((256, 256), lambda i: (0, 0)),
            pl.BlockSpec((1, 256), lambda i: (0, 0)),
            pl.BlockSpec((256, 128), lambda i: (0, 0)),
            pl.BlockSpec((bm, 1), lambda i: (i, 0)),
            pl.BlockSpec((bm, 1), lambda i: (i, 0)),
        ],
        out_specs=pl.BlockSpec((bm, 128), lambda i: (i, 0)),
        out_shape=jax.ShapeDtypeStruct((n, 128), jnp.float32),
    )(p2, w2, b2, w3, nd, ns)


def _out_call(p3, b3, nd, n):
    bm = 2000

    def body(p_ref, b_ref, nd_ref, o_ref):
        p = p_ref[...]
        o_ref[...] = (p[0] + p[1]) * nd_ref[...] + b_ref[...]

    return pl.pallas_call(
        body,
        grid=(n // bm,),
        in_specs=[
            pl.BlockSpec((2, bm, 128), lambda i: (0, i, 0)),
            pl.BlockSpec((1, 128), lambda i: (0, 0)),
            pl.BlockSpec((bm, 1), lambda i: (i, 0)),
        ],
        out_specs=pl.BlockSpec((bm, 128), lambda i: (i, 0)),
        out_shape=jax.ShapeDtypeStruct((n, 128), jnp.float32),
    )(p3, b3, nd)


def kernel(input_features, edge_index, W1, b1, W2, b2, W3, b3):
    n, d_in = input_features.shape
    e = edge_index.shape[1]
    npad = ((n + 255) // 256) * 256

    src = edge_index[0]
    dst = edge_index[1]
    srcr = src.reshape(e // C, C)
    dstr = dst.reshape(e // C, C)
    srcoff = jnp.concatenate([src, src + n]).reshape(2 * (e // C), C)
    zpad = jnp.zeros((npad,), jnp.float32)
    ztab = jnp.zeros((npad, 128), jnp.float32)
    b1r = b1.reshape(1, -1)
    b2r = b2.reshape(1, -1)
    b3r = b3.reshape(1, -1)

    degp = _deg_call(srcr, dstr, zpad, n, npad).reshape(2, 2, npad, 1)
    x1s, ns, nd = _prep_call(input_features, degp, n, npad)

    p1 = _mp_call(x1s, srcr, dstr, ztab, n, npad, split_cols=False)
    p1 = p1.reshape(2, npad, 128)[:, :n]
    x2 = _mm1_call(p1, W1, b1r, ns, nd, n).reshape(2 * n, 128)

    p2 = _mp_call(x2, srcoff, dstr, ztab, n, npad, split_cols=True)
    p2 = p2.reshape(2, npad, 128)[:, :n]
    x3 = _mm2_call(p2, W2, b2r, W3, ns, nd, n)

    p3 = _mp_call(x3, srcr, dstr, ztab, n, npad, split_cols=False)
    p3 = p3.reshape(2, npad, 128)[:, :n]
    return _out_call(p3, b3r, nd, n)
